# Initial kernel scaffold; baseline (speedup 1.0000x reference)
import jax
import jax.numpy as jnp
from jax.experimental import pallas as pl


def kernel(x, W_enc, b_enc):
    raise NotImplementedError("write your pallas kernel here")


# speedup vs baseline: 6.6512x; 6.6512x over previous
import jax
import jax.numpy as jnp
from jax.experimental import pallas as pl
from jax.experimental.pallas import tpu as pltpu

TOPK = 64
R_BLK = 128
C_BLK = 1024
CHUNK = 2048

INT_MIN = -2147483648


def _monotone_key(v):
    bits = jax.lax.bitcast_convert_type(v, jnp.int32)
    return jnp.where(bits < 0, bits ^ jnp.int32(0x7FFFFFFF), bits)


def _body(x_ref, w_ref, b_ref, o_ref, key_ref):
    j = pl.program_id(1)
    nj = pl.num_programs(1)

    acc = jax.lax.dot_general(
        x_ref[...], w_ref[...],
        dimension_numbers=(((1,), (1,)), ((), ())),
        preferred_element_type=jnp.float32,
    )
    o_ref[:, pl.ds(j * C_BLK, C_BLK)] = acc + b_ref[0, pl.ds(j * C_BLK, C_BLK)][None, :]

    @pl.when(j == nj - 1)
    def _finish():
        d_sae = o_ref.shape[1]
        n_chunks = d_sae // CHUNK

        def key_chunk(c, _):
            key_ref[:, pl.ds(c * CHUNK, CHUNK)] = _monotone_key(
                o_ref[:, pl.ds(c * CHUNK, CHUNK)])
            return 0
        jax.lax.fori_loop(0, n_chunks, key_chunk, 0)

        def count_ge(cand):
            def cbody(c, acc):
                blk = key_ref[:, pl.ds(c * CHUNK, CHUNK)]
                return acc + jnp.sum((blk >= cand).astype(jnp.int32), axis=1,
                                     keepdims=True)
            return jax.lax.fori_loop(
                0, n_chunks, cbody,
                jnp.zeros((o_ref.shape[0], 1), jnp.int32))

        def bbody(i, t):
            shift = (jnp.int32(31) - i).astype(jnp.uint32)
            cand = t + (jnp.int32(1) << shift)
            cnt = count_ge(cand)
            return jnp.where(cnt >= TOPK, cand, t)

        t = jax.lax.fori_loop(
            0, 32, bbody,
            jnp.full((o_ref.shape[0], 1), INT_MIN, jnp.int32))

        def mask_chunk(c, _):
            sl = pl.ds(c * CHUNK, CHUNK)
            o_ref[:, sl] = jnp.where(key_ref[:, sl] >= t, o_ref[:, sl], 0.0)
            return 0
        jax.lax.fori_loop(0, n_chunks, mask_chunk, 0)


def kernel(x, W_enc, b_enc):
    n_tok, d_model = x.shape
    d_sae = W_enc.shape[0]
    b2 = b_enc.reshape(1, d_sae)
    xb = x.astype(jnp.bfloat16)
    wb = W_enc.astype(jnp.bfloat16)
    grid = (n_tok // R_BLK, d_sae // C_BLK)
    return pl.pallas_call(
        _body,
        grid=grid,
        in_specs=[
            pl.BlockSpec((R_BLK, d_model), lambda i, j: (i, 0)),
            pl.BlockSpec((C_BLK, d_model), lambda i, j: (j, 0)),
            pl.BlockSpec((1, d_sae), lambda i, j: (0, 0)),
        ],
        out_specs=pl.BlockSpec((R_BLK, d_sae), lambda i, j: (i, 0)),
        out_shape=jax.ShapeDtypeStruct((n_tok, d_sae), jnp.float32),
        scratch_shapes=[pltpu.VMEM((R_BLK, d_sae), jnp.int32)],
        compiler_params=pltpu.CompilerParams(
            dimension_semantics=("arbitrary", "arbitrary"),
        ),
    )(xb, wb, b2)


# speedup vs baseline: 10.0907x; 1.5171x over previous
import jax
import jax.numpy as jnp
from jax.experimental import pallas as pl
from jax.experimental.pallas import tpu as pltpu

TOPK = 64
R_BLK = 256
C_BLK = 2048
CHUNK = 2048

INT_MIN = -2147483648
NEG_FINITE_MIN = INT_MIN + 0x800000


def _decode(key):
    bits = jnp.where(key < 0, key ^ jnp.int32(0x7FFFFFFF), key)
    return jax.lax.bitcast_convert_type(bits, jnp.float32)


def _body(x_ref, w_ref, b_ref, o_ref):
    j = pl.program_id(1)
    nj = pl.num_programs(1)

    acc = jax.lax.dot_general(
        x_ref[...], w_ref[...],
        dimension_numbers=(((1,), (1,)), ((), ())),
        preferred_element_type=jnp.float32,
    )
    o_ref[:, pl.ds(j * C_BLK, C_BLK)] = acc + b_ref[0, pl.ds(j * C_BLK, C_BLK)][None, :]

    @pl.when(j == nj - 1)
    def _finish():
        rows, d_sae = o_ref.shape
        n_chunks = d_sae // CHUNK

        def count_ge(candf):
            def cbody(c, acc):
                blk = o_ref[:, pl.ds(c * CHUNK, CHUNK)]
                return acc + jnp.sum((blk >= candf).astype(jnp.int32), axis=1,
                                     keepdims=True)
            return jax.lax.fori_loop(0, n_chunks, cbody,
                                     jnp.zeros((rows, 1), jnp.int32))

        def bbody(i, t):
            shift = (jnp.int32(31) - i).astype(jnp.uint32)
            cand = t + (jnp.int32(1) << shift)
            cnt = count_ge(_decode(cand))
            cnt = jnp.where(cand < jnp.int32(NEG_FINITE_MIN),
                            jnp.int32(d_sae), cnt)
            return jnp.where(cnt >= TOPK, cand, t)

        t = jax.lax.fori_loop(
            0, 32, bbody, jnp.full((rows, 1), INT_MIN, jnp.int32))
        tf = _decode(t)

        def mask_chunk(c, _):
            sl = pl.ds(c * CHUNK, CHUNK)
            blk = o_ref[:, sl]
            o_ref[:, sl] = jnp.where(blk >= tf, blk, 0.0)
            return 0
        jax.lax.fori_loop(0, n_chunks, mask_chunk, 0)


def kernel(x, W_enc, b_enc):
    n_tok, d_model = x.shape
    d_sae = W_enc.shape[0]
    b2 = b_enc.reshape(1, d_sae)
    xb = x.astype(jnp.bfloat16)
    wb = W_enc.astype(jnp.bfloat16)
    grid = (n_tok // R_BLK, d_sae // C_BLK)
    return pl.pallas_call(
        _body,
        grid=grid,
        in_specs=[
            pl.BlockSpec((R_BLK, d_model), lambda i, j: (i, 0)),
            pl.BlockSpec((C_BLK, d_model), lambda i, j: (j, 0)),
            pl.BlockSpec((1, d_sae), lambda i, j: (0, 0)),
        ],
        out_specs=pl.BlockSpec((R_BLK, d_sae), lambda i, j: (i, 0)),
        out_shape=jax.ShapeDtypeStruct((n_tok, d_sae), jnp.float32),
        compiler_params=pltpu.CompilerParams(
            dimension_semantics=("arbitrary", "arbitrary"),
        ),
    )(xb, wb, b2)
